# baseline (speedup 1.0000x reference)
import jax
import jax.numpy as jnp
from jax import lax
from jax.experimental import pallas as pl
from jax.experimental.pallas import tpu as pltpu
from jax.experimental.pallas import tpu_sc as plsc

N = 500000
V = 100
D = 128
NC = 2
NS = 16
NW = NC * NS
C = 400
H = C // 2
K = N // C
ITERS = (K + NW - 1) // NW


def _embed_body(emb_hbm, z_hbm, out_hbm, table_sp,
                idx0, idx1, rows0, rows1,
                sem_ga, sem_gb, sem_i0, sem_i1, sem_o0, sem_o1):
    cid = lax.axis_index("c")
    sid = lax.axis_index("s")
    wid = sid * NC + cid

    @pl.when(sid == 0)
    def _():
        pltpu.sync_copy(emb_hbm, table_sp)

    plsc.subcore_barrier()

    pltpu.async_copy(z_hbm.at[pl.ds(wid * C, C)], idx0, sem_i0)
    pltpu.async_copy(z_hbm.at[pl.ds((wid + NW) * C, C)], idx1, sem_i1)

    def do_iter(i, idx_v, rows_v, sem_i, sem_o):
        k = wid + i * NW

        @pl.when(k < K)
        def _():
            pltpu.make_async_copy(z_hbm.at[pl.ds(0, C)], idx_v, sem_i).wait()

            @pl.when(i >= 2)
            def _():
                pltpu.make_async_copy(
                    rows_v, out_hbm.at[pl.ds(0, C)], sem_o).wait()

            pltpu.async_copy(
                table_sp.at[idx_v.at[pl.ds(0, H)]],
                rows_v.at[pl.ds(0, H)], sem_ga, priority=1)
            pltpu.async_copy(
                table_sp.at[idx_v.at[pl.ds(H, H)]],
                rows_v.at[pl.ds(H, H)], sem_gb, priority=1)
            pltpu.make_async_copy(
                table_sp.at[idx_v.at[pl.ds(0, H)]],
                rows_v.at[pl.ds(0, H)], sem_ga).wait()
            pltpu.make_async_copy(
                table_sp.at[idx_v.at[pl.ds(H, H)]],
                rows_v.at[pl.ds(H, H)], sem_gb).wait()

            pltpu.async_copy(rows_v, out_hbm.at[pl.ds(k * C, C)], sem_o)

            @pl.when(k + 2 * NW < K)
            def _():
                pltpu.async_copy(
                    z_hbm.at[pl.ds((k + 2 * NW) * C, C)], idx_v, sem_i)

    def loop_body(i2, carry):
        do_iter(2 * i2, idx0, rows0, sem_i0, sem_o0)
        do_iter(2 * i2 + 1, idx1, rows1, sem_i1, sem_o1)
        return carry

    lax.fori_loop(0, ITERS // 2, loop_body, 0)

    pltpu.make_async_copy(rows0, out_hbm.at[pl.ds(0, C)], sem_o0).wait()
    pltpu.make_async_copy(rows1, out_hbm.at[pl.ds(0, C)], sem_o1).wait()


_mesh = plsc.VectorSubcoreMesh(
    core_axis_name="c", subcore_axis_name="s", num_cores=NC, num_subcores=NS
)

_embed = pl.kernel(
    _embed_body,
    out_type=jax.ShapeDtypeStruct((N, D), jnp.float32),
    mesh=_mesh,
    scratch_types=[
        pltpu.VMEM_SHARED((V, D), jnp.float32),
        pltpu.VMEM((C,), jnp.int32),
        pltpu.VMEM((C,), jnp.int32),
        pltpu.VMEM((C, D), jnp.float32),
        pltpu.VMEM((C, D), jnp.float32),
        pltpu.SemaphoreType.DMA,
        pltpu.SemaphoreType.DMA,
        pltpu.SemaphoreType.DMA,
        pltpu.SemaphoreType.DMA,
        pltpu.SemaphoreType.DMA,
        pltpu.SemaphoreType.DMA,
    ],
)


@jax.jit
def kernel(dR, Z, embeddings):
    del dR
    return _embed(embeddings, Z.astype(jnp.int32))


# speedup vs baseline: 1.0082x; 1.0082x over previous
import jax
import jax.numpy as jnp
from jax import lax
from jax.experimental import pallas as pl
from jax.experimental.pallas import tpu as pltpu
from jax.experimental.pallas import tpu_sc as plsc

N = 500000
V = 100
D = 128
NC = 2
NS = 16
NW = NC * NS
C = 400
K_MAIN = 1248
ITERS = K_MAIN // NW
CT = 200
TAIL_BASE = K_MAIN * C


def _embed_body(emb_hbm, z_hbm, out_hbm, table_sp,
                idx0, idx1, rows0, rows1,
                sem_g, sem_i0, sem_i1, sem_o0, sem_o1):
    cid = lax.axis_index("c")
    sid = lax.axis_index("s")
    wid = sid * NC + cid

    pltpu.async_copy(z_hbm.at[pl.ds(wid * C, C)], idx0, sem_i0)
    pltpu.async_copy(z_hbm.at[pl.ds((wid + NW) * C, C)], idx1, sem_i1)

    @pl.when(sid == 0)
    def _():
        pltpu.sync_copy(emb_hbm, table_sp)

    plsc.subcore_barrier()

    def do_iter(i, idx_v, rows_v, sem_i, sem_o):
        k = wid + i * NW

        pltpu.make_async_copy(z_hbm.at[pl.ds(0, C)], idx_v, sem_i).wait()

        @pl.when(i >= 2)
        def _():
            pltpu.make_async_copy(
                rows_v, out_hbm.at[pl.ds(0, C)], sem_o).wait()

        pltpu.async_copy(table_sp.at[idx_v], rows_v, sem_g)
        pltpu.make_async_copy(table_sp.at[idx_v], rows_v, sem_g).wait()

        pltpu.async_copy(rows_v, out_hbm.at[pl.ds(k * C, C)], sem_o)

        @pl.when(i + 2 < ITERS)
        def _():
            pltpu.async_copy(
                z_hbm.at[pl.ds((k + 2 * NW) * C, C)], idx_v, sem_i)

    def loop_body(i2, carry):
        do_iter(2 * i2, idx0, rows0, sem_i0, sem_o0)
        do_iter(2 * i2 + 1, idx1, rows1, sem_i1, sem_o1)
        return carry

    lax.fori_loop(0, ITERS // 2, loop_body, 0)

    do_iter(ITERS - 1, idx0, rows0, sem_i0, sem_o0)

    @pl.when(wid < 4)
    def _():
        base = TAIL_BASE + wid * CT
        pltpu.async_copy(
            z_hbm.at[pl.ds(base, CT)], idx1.at[pl.ds(0, CT)], sem_i1)
        pltpu.make_async_copy(
            z_hbm.at[pl.ds(0, CT)], idx1.at[pl.ds(0, CT)], sem_i1).wait()
        pltpu.make_async_copy(
            rows1, out_hbm.at[pl.ds(0, C)], sem_o1).wait()
        pltpu.async_copy(
            table_sp.at[idx1.at[pl.ds(0, CT)]],
            rows1.at[pl.ds(0, CT)], sem_g)
        pltpu.make_async_copy(
            table_sp.at[idx1.at[pl.ds(0, CT)]],
            rows1.at[pl.ds(0, CT)], sem_g).wait()
        pltpu.async_copy(
            rows1.at[pl.ds(0, CT)], out_hbm.at[pl.ds(base, CT)], sem_o1)
        pltpu.make_async_copy(
            rows1.at[pl.ds(0, CT)], out_hbm.at[pl.ds(base, CT)], sem_o1).wait()

    @pl.when(jnp.logical_not(wid < 4))
    def _():
        pltpu.make_async_copy(
            rows1, out_hbm.at[pl.ds(0, C)], sem_o1).wait()

    pltpu.make_async_copy(rows0, out_hbm.at[pl.ds(0, C)], sem_o0).wait()


_mesh = plsc.VectorSubcoreMesh(
    core_axis_name="c", subcore_axis_name="s", num_cores=NC, num_subcores=NS
)

_embed = pl.kernel(
    _embed_body,
    out_type=jax.ShapeDtypeStruct((N, D), jnp.float32),
    mesh=_mesh,
    scratch_types=[
        pltpu.VMEM_SHARED((V, D), jnp.float32),
        pltpu.VMEM((C,), jnp.int32),
        pltpu.VMEM((C,), jnp.int32),
        pltpu.VMEM((C, D), jnp.float32),
        pltpu.VMEM((C, D), jnp.float32),
        pltpu.SemaphoreType.DMA,
        pltpu.SemaphoreType.DMA,
        pltpu.SemaphoreType.DMA,
        pltpu.SemaphoreType.DMA,
        pltpu.SemaphoreType.DMA,
    ],
)


@jax.jit
def kernel(dR, Z, embeddings):
    del dR
    return _embed(embeddings, Z.astype(jnp.int32))


# speedup vs baseline: 1.0106x; 1.0023x over previous
import jax
import jax.numpy as jnp
from jax import lax
from jax.experimental import pallas as pl
from jax.experimental.pallas import tpu as pltpu
from jax.experimental.pallas import tpu_sc as plsc

N = 500000
V = 100
D = 128
NC = 2
NS = 16
NW = NC * NS
C = 504
K_MAIN = 992
ITERS = K_MAIN // NW
CT = 32
TAIL_BASE = K_MAIN * C


def _embed_body(emb_hbm, z_hbm, out_hbm, table_sp,
                idx0, idx1, rows0, rows1,
                sem_g, sem_i0, sem_i1, sem_o0, sem_o1):
    cid = lax.axis_index("c")
    sid = lax.axis_index("s")
    wid = sid * NC + cid

    pltpu.async_copy(z_hbm.at[pl.ds(wid * C, C)], idx0, sem_i0)
    pltpu.async_copy(z_hbm.at[pl.ds((wid + NW) * C, C)], idx1, sem_i1)

    @pl.when(sid == 0)
    def _():
        pltpu.sync_copy(emb_hbm, table_sp)

    plsc.subcore_barrier()

    def do_iter(i, idx_v, rows_v, sem_i, sem_o):
        k = wid + i * NW

        pltpu.make_async_copy(z_hbm.at[pl.ds(0, C)], idx_v, sem_i).wait()

        @pl.when(i >= 2)
        def _():
            pltpu.make_async_copy(
                rows_v, out_hbm.at[pl.ds(0, C)], sem_o).wait()

        pltpu.async_copy(table_sp.at[idx_v], rows_v, sem_g)
        pltpu.make_async_copy(table_sp.at[idx_v], rows_v, sem_g).wait()

        pltpu.async_copy(rows_v, out_hbm.at[pl.ds(k * C, C)], sem_o)

        @pl.when(i + 2 < ITERS)
        def _():
            pltpu.async_copy(
                z_hbm.at[pl.ds((k + 2 * NW) * C, C)], idx_v, sem_i)

    def loop_body(i2, carry):
        do_iter(2 * i2, idx0, rows0, sem_i0, sem_o0)
        do_iter(2 * i2 + 1, idx1, rows1, sem_i1, sem_o1)
        return carry

    lax.fori_loop(0, ITERS // 2, loop_body, 0)

    do_iter(ITERS - 1, idx0, rows0, sem_i0, sem_o0)

    @pl.when(wid < 1)
    def _():
        base = TAIL_BASE + wid * CT
        pltpu.async_copy(
            z_hbm.at[pl.ds(base, CT)], idx1.at[pl.ds(0, CT)], sem_i1)
        pltpu.make_async_copy(
            z_hbm.at[pl.ds(0, CT)], idx1.at[pl.ds(0, CT)], sem_i1).wait()
        pltpu.make_async_copy(
            rows1, out_hbm.at[pl.ds(0, C)], sem_o1).wait()
        pltpu.async_copy(
            table_sp.at[idx1.at[pl.ds(0, CT)]],
            rows1.at[pl.ds(0, CT)], sem_g)
        pltpu.make_async_copy(
            table_sp.at[idx1.at[pl.ds(0, CT)]],
            rows1.at[pl.ds(0, CT)], sem_g).wait()
        pltpu.async_copy(
            rows1.at[pl.ds(0, CT)], out_hbm.at[pl.ds(base, CT)], sem_o1)
        pltpu.make_async_copy(
            rows1.at[pl.ds(0, CT)], out_hbm.at[pl.ds(base, CT)], sem_o1).wait()

    @pl.when(jnp.logical_not(wid < 1))
    def _():
        pltpu.make_async_copy(
            rows1, out_hbm.at[pl.ds(0, C)], sem_o1).wait()

    pltpu.make_async_copy(rows0, out_hbm.at[pl.ds(0, C)], sem_o0).wait()


_mesh = plsc.VectorSubcoreMesh(
    core_axis_name="c", subcore_axis_name="s", num_cores=NC, num_subcores=NS
)

_embed = pl.kernel(
    _embed_body,
    out_type=jax.ShapeDtypeStruct((N, D), jnp.float32),
    mesh=_mesh,
    scratch_types=[
        pltpu.VMEM_SHARED((V, D), jnp.float32),
        pltpu.VMEM((C,), jnp.int32),
        pltpu.VMEM((C,), jnp.int32),
        pltpu.VMEM((C, D), jnp.float32),
        pltpu.VMEM((C, D), jnp.float32),
        pltpu.SemaphoreType.DMA,
        pltpu.SemaphoreType.DMA,
        pltpu.SemaphoreType.DMA,
        pltpu.SemaphoreType.DMA,
        pltpu.SemaphoreType.DMA,
    ],
)


@jax.jit
def kernel(dR, Z, embeddings):
    del dR
    return _embed(embeddings, Z.astype(jnp.int32))
